# baseline (speedup 1.0000x reference)
import functools
import jax
import jax.numpy as jnp
from jax.experimental import pallas as pl
from jax.experimental.pallas import tpu as pltpu

_B, _N, _M, _C = 16, 20000, 100, 80
_MP = 128
_CW = 2048
_G = (_N + _CW - 1) // _CW
_L_COORD, _L_OBJ, _L_NOOBJ, _L_CLS = 5.0, 1.0, 0.5, 1.0


def _loss_kernel(bbox_ref, obj_ref, cls_ref, gtb_ref, lbl_ref, out_ref,
                 acc_ref):
    i = pl.program_id(0)

    x = obj_ref[...]
    col = jax.lax.broadcasted_iota(jnp.int32, (_B, _CW), 1) + i * _CW
    valid = col < _N
    obj_pos = col < _M
    t = jnp.log1p(jnp.exp(-jnp.abs(x)))
    sp_neg_x = t + jnp.maximum(-x, 0.0)
    sp_pos_x = t + jnp.maximum(x, 0.0)
    pos_sum = jnp.sum(jnp.where(obj_pos, sp_neg_x, 0.0))
    neg_sum = jnp.sum(jnp.where(valid & ~obj_pos, sp_pos_x, 0.0))

    @pl.when(i == 0)
    def _init():
        acc_ref[0] = pos_sum
        acc_ref[1] = neg_sum

    @pl.when(i > 0)
    def _acc():
        acc_ref[0] += pos_sum
        acc_ref[1] += neg_sum

    @pl.when(i == 0)
    def _dense():
        pb = bbox_ref[:, : _M, :]
        gb = gtb_ref[...]
        px, py, pw, ph = pb[:, :, 0], pb[:, :, 1], pb[:, :, 2], pb[:, :, 3]
        gx, gy, gw, gh = gb[:, :, 0], gb[:, :, 1], gb[:, :, 2], gb[:, :, 3]
        px1, px2 = px - pw * 0.5, px + pw * 0.5
        py1, py2 = py - ph * 0.5, py + ph * 0.5
        gx1, gx2 = gx - gw * 0.5, gx + gw * 0.5
        gy1, gy2 = gy - gh * 0.5, gy + gh * 0.5
        iw = jnp.maximum(jnp.minimum(px2, gx2) - jnp.maximum(px1, gx1), 0.0)
        ih = jnp.maximum(jnp.minimum(py2, gy2) - jnp.maximum(py1, gy1), 0.0)
        inter = iw * ih
        union = (px2 - px1) * (py2 - py1) + (gx2 - gx1) * (gy2 - gy1) - inter
        iou = inter / (union + 1e-07)
        ew = jnp.maximum(px2, gx2) - jnp.minimum(px1, gx1)
        eh = jnp.maximum(py2, gy2) - jnp.minimum(py1, gy1)
        enclose = ew * eh
        giou = 1.0 - (iou - (enclose - union) / (enclose + 1e-07))
        out_ref[1] = jnp.sum(giou) * (_L_COORD / (_B * _M))

        z = cls_ref[:, : _M, :]
        mx = jnp.max(z, axis=-1)
        lse = mx + jnp.log(jnp.sum(jnp.exp(z - mx[:, :, None]), axis=-1))
        lab = lbl_ref[...]
        cls_iota = jax.lax.broadcasted_iota(jnp.int32, (_B, _M, _C), 2)
        z_lab = jnp.sum(
            jnp.where(cls_iota == lab[:, :, None], z, 0.0), axis=-1
        )
        out_ref[3] = jnp.sum(lse - z_lab) * (_L_CLS / (_B * _M))

    @pl.when(i == _G - 1)
    def _fini():
        loss_obj = acc_ref[0] * (_L_OBJ / (_B * _M)) + acc_ref[1] * (
            _L_NOOBJ / (_B * (_N - _M))
        )
        out_ref[2] = loss_obj
        out_ref[0] = out_ref[1] + loss_obj + out_ref[3]


def kernel(pred_bbox, pred_obj, pred_cls, gt_boxes, gt_labels):
    bbox_s = pred_bbox[:, :_MP, :]
    cls_s = pred_cls[:, :_MP, :]
    lbl = gt_labels.astype(jnp.int32)
    out = pl.pallas_call(
        _loss_kernel,
        out_shape=jax.ShapeDtypeStruct((4,), jnp.float32),
        grid=(_G,),
        in_specs=[
            pl.BlockSpec((_B, _MP, 4), lambda i: (0, 0, 0)),
            pl.BlockSpec((_B, _CW), lambda i: (0, i)),
            pl.BlockSpec((_B, _MP, _C), lambda i: (0, 0, 0)),
            pl.BlockSpec((_B, _M, 4), lambda i: (0, 0, 0)),
            pl.BlockSpec((_B, _M), lambda i: (0, 0)),
        ],
        out_specs=pl.BlockSpec(memory_space=pltpu.SMEM),
        scratch_shapes=[pltpu.SMEM((2,), jnp.float32)],
    )(bbox_s, pred_obj, cls_s, gt_boxes, lbl)
    return (out[0], out[1], out[2], out[3])


# speedup vs baseline: 1.3759x; 1.3759x over previous
import jax
import jax.numpy as jnp
from jax.experimental import pallas as pl
from jax.experimental.pallas import tpu as pltpu

_B, _N, _M, _C = 16, 20000, 100, 80
_MP = 128
_L_COORD, _L_OBJ, _L_NOOBJ, _L_CLS = 5.0, 1.0, 0.5, 1.0


def _loss_kernel(bbox_ref, obj_ref, cls_ref, gtb_ref, lbl_ref,
                 tot_ref, bb_ref, ob_ref, cl_ref):
    pb = bbox_ref[:, : _M, :]
    gb = gtb_ref[...]
    px, py, pw, ph = pb[:, :, 0], pb[:, :, 1], pb[:, :, 2], pb[:, :, 3]
    gx, gy, gw, gh = gb[:, :, 0], gb[:, :, 1], gb[:, :, 2], gb[:, :, 3]
    px1, px2 = px - pw * 0.5, px + pw * 0.5
    py1, py2 = py - ph * 0.5, py + ph * 0.5
    gx1, gx2 = gx - gw * 0.5, gx + gw * 0.5
    gy1, gy2 = gy - gh * 0.5, gy + gh * 0.5
    iw = jnp.maximum(jnp.minimum(px2, gx2) - jnp.maximum(px1, gx1), 0.0)
    ih = jnp.maximum(jnp.minimum(py2, gy2) - jnp.maximum(py1, gy1), 0.0)
    inter = iw * ih
    union = (px2 - px1) * (py2 - py1) + (gx2 - gx1) * (gy2 - gy1) - inter
    iou = inter / (union + 1e-07)
    ew = jnp.maximum(px2, gx2) - jnp.minimum(px1, gx1)
    eh = jnp.maximum(py2, gy2) - jnp.minimum(py1, gy1)
    enclose = ew * eh
    giou = 1.0 - (iou - (enclose - union) / (enclose + 1e-07))
    loss_bbox = jnp.sum(giou) * (_L_COORD / (_B * _M))

    x = obj_ref[...]
    col = jax.lax.broadcasted_iota(jnp.int32, (_B, _N), 1)
    obj_pos = col < _M
    t = jnp.log1p(jnp.exp(-jnp.abs(x)))
    sp_neg_x = t + jnp.maximum(-x, 0.0)
    sp_pos_x = t + jnp.maximum(x, 0.0)
    pos_sum = jnp.sum(jnp.where(obj_pos, sp_neg_x, 0.0))
    neg_sum = jnp.sum(jnp.where(obj_pos, 0.0, sp_pos_x))
    loss_obj = pos_sum * (_L_OBJ / (_B * _M)) + neg_sum * (
        _L_NOOBJ / (_B * (_N - _M))
    )

    z = cls_ref[:, : _M, :]
    m = jnp.max(z, axis=-1)
    lse = m + jnp.log(jnp.sum(jnp.exp(z - m[:, :, None]), axis=-1))
    lab = lbl_ref[...]
    cls_iota = jax.lax.broadcasted_iota(jnp.int32, (_B, _M, _C), 2)
    z_lab = jnp.sum(jnp.where(cls_iota == lab[:, :, None], z, 0.0), axis=-1)
    nll = lse - z_lab
    loss_cls = jnp.sum(nll) * (_L_CLS / (_B * _M))

    tot_ref[0] = loss_bbox + loss_obj + loss_cls
    bb_ref[0] = loss_bbox
    ob_ref[0] = loss_obj
    cl_ref[0] = loss_cls


def kernel(pred_bbox, pred_obj, pred_cls, gt_boxes, gt_labels):
    bbox_s = pred_bbox[:, :_MP, :]
    cls_s = pred_cls[:, :_MP, :]
    lbl = gt_labels.astype(jnp.int32)
    scalar = jax.ShapeDtypeStruct((1,), jnp.float32)
    smem = pl.BlockSpec(memory_space=pltpu.SMEM)
    tot, bb, ob, cl = pl.pallas_call(
        _loss_kernel,
        out_shape=(scalar, scalar, scalar, scalar),
        grid=(1,),
        in_specs=[
            pl.BlockSpec((_B, _MP, 4), lambda i: (0, 0, 0)),
            pl.BlockSpec((_B, _N), lambda i: (0, 0)),
            pl.BlockSpec((_B, _MP, _C), lambda i: (0, 0, 0)),
            pl.BlockSpec((_B, _M, 4), lambda i: (0, 0, 0)),
            pl.BlockSpec((_B, _M), lambda i: (0, 0)),
        ],
        out_specs=(smem, smem, smem, smem),
    )(bbox_s, pred_obj, cls_s, gt_boxes, lbl)
    return (tot[0], bb[0], ob[0], cl[0])


# speedup vs baseline: 1.9421x; 1.4115x over previous
import jax
import jax.numpy as jnp
from jax.experimental import pallas as pl
from jax.experimental.pallas import tpu as pltpu

_B, _N, _M, _C = 16, 20000, 100, 80
_MP = 128
_L_COORD, _L_OBJ, _L_NOOBJ, _L_CLS = 5.0, 1.0, 0.5, 1.0


def _loss_kernel(bbox_ref, obj_ref, cls_ref, gtb_ref, lbl_ref,
                 tot_ref, bb_ref, ob_ref, cl_ref):
    px, py = bbox_ref[0][:, : _M], bbox_ref[1][:, : _M]
    pw, ph = bbox_ref[2][:, : _M], bbox_ref[3][:, : _M]
    gx, gy, gw, gh = gtb_ref[0], gtb_ref[1], gtb_ref[2], gtb_ref[3]
    px1, px2 = px - pw * 0.5, px + pw * 0.5
    py1, py2 = py - ph * 0.5, py + ph * 0.5
    gx1, gx2 = gx - gw * 0.5, gx + gw * 0.5
    gy1, gy2 = gy - gh * 0.5, gy + gh * 0.5
    iw = jnp.maximum(jnp.minimum(px2, gx2) - jnp.maximum(px1, gx1), 0.0)
    ih = jnp.maximum(jnp.minimum(py2, gy2) - jnp.maximum(py1, gy1), 0.0)
    inter = iw * ih
    union = (px2 - px1) * (py2 - py1) + (gx2 - gx1) * (gy2 - gy1) - inter
    iou = inter / (union + 1e-07)
    ew = jnp.maximum(px2, gx2) - jnp.minimum(px1, gx1)
    eh = jnp.maximum(py2, gy2) - jnp.minimum(py1, gy1)
    enclose = ew * eh
    giou = 1.0 - (iou - (enclose - union) / (enclose + 1e-07))
    loss_bbox = jnp.sum(giou) * (_L_COORD / (_B * _M))

    x = obj_ref[...]
    t = jnp.log1p(jnp.exp(-jnp.abs(x)))
    all_sum = jnp.sum(t + jnp.maximum(x, 0.0))
    xs = x[:, : _M]
    ts = t[:, : _M]
    pos_sum = jnp.sum(ts + jnp.maximum(-xs, 0.0))
    over_sum = jnp.sum(ts + jnp.maximum(xs, 0.0))
    loss_obj = pos_sum * (_L_OBJ / (_B * _M)) + (all_sum - over_sum) * (
        _L_NOOBJ / (_B * (_N - _M))
    )

    z = cls_ref[:, : _M, :]
    m = jnp.max(z, axis=-1)
    lse = m + jnp.log(jnp.sum(jnp.exp(z - m[:, :, None]), axis=-1))
    lab = lbl_ref[...]
    cls_iota = jax.lax.broadcasted_iota(jnp.int32, (_B, _M, _C), 2)
    z_lab = jnp.sum(jnp.where(cls_iota == lab[:, :, None], z, 0.0), axis=-1)
    nll = lse - z_lab
    loss_cls = jnp.sum(nll) * (_L_CLS / (_B * _M))

    tot_ref[0] = loss_bbox + loss_obj + loss_cls
    bb_ref[0] = loss_bbox
    ob_ref[0] = loss_obj
    cl_ref[0] = loss_cls


def kernel(pred_bbox, pred_obj, pred_cls, gt_boxes, gt_labels):
    bbox_t = jnp.transpose(pred_bbox[:, :_MP, :], (2, 0, 1))
    gt_t = jnp.transpose(gt_boxes, (2, 0, 1))
    cls_s = pred_cls[:, :_MP, :]
    lbl = gt_labels.astype(jnp.int32)
    scalar = jax.ShapeDtypeStruct((1,), jnp.float32)
    smem = pl.BlockSpec(memory_space=pltpu.SMEM)
    tot, bb, ob, cl = pl.pallas_call(
        _loss_kernel,
        out_shape=(scalar, scalar, scalar, scalar),
        grid=(1,),
        in_specs=[
            pl.BlockSpec((4, _B, _MP), lambda i: (0, 0, 0)),
            pl.BlockSpec((_B, _N), lambda i: (0, 0)),
            pl.BlockSpec((_B, _MP, _C), lambda i: (0, 0, 0)),
            pl.BlockSpec((4, _B, _M), lambda i: (0, 0, 0)),
            pl.BlockSpec((_B, _M), lambda i: (0, 0)),
        ],
        out_specs=(smem, smem, smem, smem),
    )(bbox_t, pred_obj, cls_s, gt_t, lbl)
    return (tot[0], bb[0], ob[0], cl[0])
